# Initial kernel scaffold; baseline (speedup 1.0000x reference)
import jax
import jax.numpy as jnp
from jax.experimental import pallas as pl


def kernel(x, params, MM, PM, K):
    raise NotImplementedError("write your pallas kernel here")


# speedup vs baseline: 1.4873x; 1.4873x over previous
import jax
import jax.numpy as jnp
from jax.experimental import pallas as pl


def _mlp(ws, h):
    for Wt, b in ws[:-1]:
        h = jnp.maximum(
            jnp.dot(h, Wt, preferred_element_type=jnp.float32) + b, 0.0)
    Wt, b = ws[-1]
    return jnp.dot(h, Wt, preferred_element_type=jnp.float32) + b


def _gnn_step(I, J, D, refs):
    x_ref, init_ref, mm_ref = refs[0], refs[1], refs[2]
    wrefs, out_ref = refs[3:-1], refs[-1]
    ws = [[(wrefs[m * 8 + 2 * l][...], wrefs[m * 8 + 2 * l + 1][...])
           for l in range(4)] for m in range(4)]

    x = x_ref[...]
    init = init_ref[...]
    mm = mm_ref[...]

    max_step = jnp.max(mm, axis=1, keepdims=True)
    pred_t = mm - 1
    succ_t = mm + 1

    pred_feat = jnp.zeros((I, J, D), jnp.float32)
    succ_feat = jnp.zeros((I, J, D), jnp.float32)
    pred_done = jnp.zeros((I, J, 1), jnp.bool_)
    succ_done = jnp.zeros((I, J, 1), jnp.bool_)
    for a in range(J):
        col = mm[:, a:a + 1, :]
        xa = x[:, a:a + 1, :]
        pm = (col == pred_t) & (~pred_done)
        sm = (col == succ_t) & (~succ_done)
        pred_feat = pred_feat + jnp.where(pm, xa, 0.0)
        succ_feat = succ_feat + jnp.where(sm, xa, 0.0)
        pred_done = pred_done | pm
        succ_done = succ_done | sm
    x0 = x[:, 0:1, :]
    pred_feat = jnp.where(pred_done, pred_feat, x0)
    succ_feat = jnp.where(succ_done, succ_feat, x0)

    has_pred = mm != 0
    has_succ = mm != max_step
    a1_in = jnp.where(has_pred, pred_feat, 0.0)
    a2_in = jnp.where(has_succ, succ_feat, 0.0)
    a3_in = jnp.sum(x, axis=0, keepdims=True) - x

    N = I * J
    xf = x.reshape(N, D)
    a4_vec = jnp.maximum(jnp.sum(xf, axis=0, keepdims=True), 0.0)

    a1 = jnp.maximum(_mlp(ws[0], a1_in.reshape(N, D)), 0.0)
    a2 = jnp.maximum(_mlp(ws[1], a2_in.reshape(N, D)), 0.0)
    a3 = jnp.maximum(_mlp(ws[2], a3_in.reshape(N, D)), 0.0)
    a4 = jnp.broadcast_to(a4_vec, (N, D))

    cat = jnp.concatenate([a1, a2, a3, a4, xf, init.reshape(N, D)], axis=-1)
    out_ref[...] = _mlp(ws[3], cat).reshape(I, J, D)


def kernel(x, params, MM, PM, K):
    del PM
    I, J, D = x.shape
    wlist = []
    for name in ('f1', 'f2', 'f3', 'f4'):
        for (W, b) in params[name]:
            wlist.append(W.T)
            wlist.append(b.reshape(1, -1))
    init = x

    step = pl.pallas_call(
        lambda *refs: _gnn_step(I, J, D, refs),
        out_shape=jax.ShapeDtypeStruct((I, J, D), jnp.float32),
    )

    MM3 = MM[:, :, None]

    def body(_, xc):
        return step(xc, init, MM3, *wlist)

    return jax.lax.fori_loop(0, K, body, x)


# speedup vs baseline: 4.2274x; 2.8423x over previous
import jax
import jax.numpy as jnp
from jax.experimental import pallas as pl


def _mlp(ws, h):
    for Wt, b in ws[:-1]:
        h = jnp.maximum(
            jnp.dot(h, Wt, preferred_element_type=jnp.float32) + b, 0.0)
    Wt, b = ws[-1]
    return jnp.dot(h, Wt, preferred_element_type=jnp.float32) + b


def _gnn_step(I, J, D, refs):
    x_ref, init_ref, mmT_ref = refs[0], refs[1], refs[2]
    wrefs, out_ref = refs[3:-1], refs[-1]
    ws = [[(wrefs[m * 8 + 2 * l][...], wrefs[m * 8 + 2 * l + 1][...])
           for l in range(4)] for m in range(4)]

    x = x_ref[...]
    init = init_ref[...]
    mmT = mmT_ref[...]

    xT = jnp.transpose(x, (1, 2, 0))
    max_T = jnp.max(mmT, axis=0, keepdims=True)
    pred_t = mmT - 1
    succ_t = mmT + 1

    pfT = jnp.zeros((J, D, I), jnp.float32)
    sfT = jnp.zeros((J, D, I), jnp.float32)
    pdone = jnp.zeros((J, 1, I), jnp.bool_)
    sdone = jnp.zeros((J, 1, I), jnp.bool_)
    for a in range(J):
        col = mmT[a:a + 1]
        xa = xT[a:a + 1]
        pm = (col == pred_t) & (~pdone)
        sm = (col == succ_t) & (~sdone)
        pfT = pfT + jnp.where(pm, xa, 0.0)
        sfT = sfT + jnp.where(sm, xa, 0.0)
        pdone = pdone | pm
        sdone = sdone | sm
    x0 = xT[0:1]
    pfT = jnp.where(pdone, pfT, x0)
    sfT = jnp.where(sdone, sfT, x0)
    a1_inT = jnp.where(mmT != 0, pfT, 0.0)
    a2_inT = jnp.where(mmT != max_T, sfT, 0.0)

    a3_in = jnp.sum(x, axis=0, keepdims=True) - x

    N = I * J
    xf = x.reshape(N, D)
    a4_vec = jnp.maximum(jnp.sum(xf, axis=0, keepdims=True), 0.0)

    a1 = jnp.maximum(
        _mlp(ws[0], jnp.transpose(a1_inT, (2, 0, 1)).reshape(N, D)), 0.0)
    a2 = jnp.maximum(
        _mlp(ws[1], jnp.transpose(a2_inT, (2, 0, 1)).reshape(N, D)), 0.0)
    a3 = jnp.maximum(_mlp(ws[2], a3_in.reshape(N, D)), 0.0)
    a4 = jnp.broadcast_to(a4_vec, (N, D))

    cat = jnp.concatenate([a1, a2, a3, a4, xf, init.reshape(N, D)], axis=-1)
    out_ref[...] = _mlp(ws[3], cat).reshape(I, J, D)


def kernel(x, params, MM, PM, K):
    del PM
    I, J, D = x.shape
    wlist = []
    for name in ('f1', 'f2', 'f3', 'f4'):
        for (W, b) in params[name]:
            wlist.append(W.T)
            wlist.append(b.reshape(1, -1))
    init = x

    step = pl.pallas_call(
        lambda *refs: _gnn_step(I, J, D, refs),
        out_shape=jax.ShapeDtypeStruct((I, J, D), jnp.float32),
    )

    MMT = MM.T[:, None, :]

    def body(_, xc):
        return step(xc, init, MMT, *wlist)

    return jax.lax.fori_loop(0, K, body, x)
